# baseline (speedup 1.0000x reference)
import functools

import jax
import jax.numpy as jnp
from jax import lax
from jax.experimental import pallas as pl
from jax.experimental.pallas import tpu as pltpu
from jax.experimental.pallas import tpu_sc as plsc

_HIDDEN = 64
_BATCH = 16384
_NITEMS = 1000000
_NC, _NS, _L = 2, 16, 16
_NW = _NC * _NS
_BPW = _BATCH // _NW
_NCHUNK = 4
_CHUNK = _BPW // _NCHUNK

_SC_CK = 512
_SC_CPW = 20
_IPW = _SC_CK * _SC_CPW
_R_SC = _NW * _IPW
_R_TC = _NITEMS - _R_SC

_BLK = 32768
_TC_BLK0 = _R_SC // _BLK
_TC_GRID = (_R_TC + _BLK - 1) // _BLK


def _project_body(w_ref, b_ref, tabT_ref, out_ref):
    out_ref[...] = jnp.dot(w_ref[...], tabT_ref[...],
                           preferred_element_type=jnp.float32)[0] + b_ref[0]


_tc_project = pl.pallas_call(
    _project_body,
    grid=(_TC_GRID,),
    in_specs=[
        pl.BlockSpec((1, _HIDDEN), lambda j: (0, 0)),
        pl.BlockSpec(memory_space=pltpu.SMEM),
        pl.BlockSpec((_HIDDEN, _BLK), lambda j: (0, j + _TC_BLK0)),
    ],
    out_specs=pl.BlockSpec((_BLK,), lambda j: (j,)),
    out_shape=jax.ShapeDtypeStruct((_R_TC,), jnp.float32),
)

_mesh = plsc.VectorSubcoreMesh(core_axis_name="c", subcore_axis_name="s")


@functools.partial(
    pl.kernel,
    mesh=_mesh,
    out_type=jax.ShapeDtypeStruct((_R_SC,), jnp.float32),
    scratch_types=[
        pltpu.VMEM((_HIDDEN,), jnp.float32),
        pltpu.VMEM((_L,), jnp.float32),
        pltpu.VMEM((2, _HIDDEN, _SC_CK), jnp.float32),
        pltpu.VMEM((_IPW,), jnp.float32),
        pltpu.SemaphoreType.DMA,
    ],
)
def _sc_project(tabT_hbm, w_hbm, b_hbm, out_hbm, w_v, b_v, buf, out_v, sem):
    wid = lax.axis_index("s") * _NC + lax.axis_index("c")
    base_i = wid * _IPW

    pltpu.sync_copy(w_hbm, w_v)
    pltpu.sync_copy(b_hbm, b_v)
    bias_vec = b_v[...]
    w_chunks = [w_v[pl.ds(k * _L, _L)] for k in range(_HIDDEN // _L)]
    iota16 = lax.iota(jnp.int32, _L)
    del iota16

    pltpu.async_copy(
        tabT_hbm.at[:, pl.ds(base_i, _SC_CK)], buf.at[0], sem)

    def chunk_body(c, carry):
        @pl.when(c + 1 < _SC_CPW)
        def _fire_next():
            pltpu.async_copy(
                tabT_hbm.at[:, pl.ds(base_i + (c + 1) * _SC_CK, _SC_CK)],
                buf.at[lax.rem(c + 1, 2)], sem)

        b = lax.rem(c, 2)
        pltpu.make_async_copy(
            tabT_hbm.at[:, pl.ds(base_i, _SC_CK)], buf.at[b], sem).wait()

        def group_body(g, inner_carry):
            base = g * _L
            acc = bias_vec
            for h in range(_HIDDEN):
                col = buf[b, h, pl.ds(base, _L)]
                acc = acc + col * w_chunks[h // _L][h % _L]
            out_v[pl.ds(c * _SC_CK + base, _L)] = acc
            return inner_carry

        lax.fori_loop(0, _SC_CK // _L, group_body, 0)
        return carry

    lax.fori_loop(0, _SC_CPW, chunk_body, 0)

    pltpu.sync_copy(out_v, out_hbm.at[pl.ds(base_i, _IPW)])


@functools.partial(
    pl.kernel,
    mesh=_mesh,
    out_type=jax.ShapeDtypeStruct((_BATCH,), jnp.float32),
    scratch_types=[
        pltpu.VMEM((_NCHUNK, _CHUNK), jnp.int32),
        pltpu.VMEM((_NCHUNK, _CHUNK), jnp.int32),
        pltpu.VMEM((_NCHUNK, _CHUNK), jnp.int32),
        pltpu.VMEM((_BPW,), jnp.float32),
        pltpu.VMEM((_BPW,), jnp.float32),
        pltpu.SemaphoreType.DMA,
    ],
)
def _sc_gather(ids_hbm, plo_hbm, phi_hbm, out_hbm,
               idx_v, idx1_v, idx2_v, v1_v, v2_v, sem):
    wid = lax.axis_index("s") * _NC + lax.axis_index("c")

    pltpu.sync_copy(ids_hbm.at[pl.ds(wid * _NCHUNK, _NCHUNK)], idx_v)

    for j in range(_NCHUNK):
        for k in range(_CHUNK // _L):
            iv = idx_v[j, pl.ds(k * _L, _L)]
            idx1_v[j, pl.ds(k * _L, _L)] = jnp.minimum(iv, _R_SC - 1)
            idx2_v[j, pl.ds(k * _L, _L)] = jnp.clip(
                iv - _R_SC, 0, _R_TC - 1)

    copies = []
    for j in range(_NCHUNK):
        copies.append(pltpu.async_copy(
            plo_hbm.at[idx1_v.at[j]],
            v1_v.at[pl.ds(j * _CHUNK, _CHUNK)], sem))
        copies.append(pltpu.async_copy(
            phi_hbm.at[idx2_v.at[j]],
            v2_v.at[pl.ds(j * _CHUNK, _CHUNK)], sem))
    for c in copies:
        c.wait()

    for j in range(_NCHUNK):
        for k in range(_CHUNK // _L):
            off = j * _CHUNK + k * _L
            m = idx_v[j, pl.ds(k * _L, _L)] < _R_SC
            v1_v[pl.ds(off, _L)] = jnp.where(
                m, v1_v[pl.ds(off, _L)], v2_v[pl.ds(off, _L)])

    pltpu.sync_copy(v1_v, out_hbm.at[pl.ds(wid * _BPW, _BPW)])


def kernel(item_ids, emb_table, fc_w, fc_b):
    ids2d = item_ids.astype(jnp.int32).reshape(_NW * _NCHUNK, _CHUNK)
    tabT = emb_table.T
    w = fc_w.astype(jnp.float32)
    b = fc_b.astype(jnp.float32)
    bias_vec = jnp.broadcast_to(b, (_L,))
    proj_lo = _sc_project(tabT, w.reshape(_HIDDEN), bias_vec)
    proj_hi = _tc_project(w, b, tabT)
    out = _sc_gather(ids2d, proj_lo, proj_hi)
    return out.reshape(_BATCH, 1)


# speedup vs baseline: 1.5980x; 1.5980x over previous
import functools

import jax
import jax.numpy as jnp
from jax import lax
from jax.experimental import pallas as pl
from jax.experimental.pallas import tpu as pltpu
from jax.experimental.pallas import tpu_sc as plsc

_HIDDEN = 64
_BATCH = 16384
_NITEMS = 1000000
_NC, _NS, _L = 2, 16, 16
_NW = _NC * _NS
_BPW = _BATCH // _NW
_NCHUNK = 4
_CHUNK = _BPW // _NCHUNK

_BLK = 32768
_GRID = (_NITEMS + _BLK - 1) // _BLK


def _project_body(w_ref, b_ref, tabT_ref, out_ref):
    out_ref[...] = jnp.dot(w_ref[...], tabT_ref[...],
                           preferred_element_type=jnp.float32)[0] + b_ref[0]


_tc_project = pl.pallas_call(
    _project_body,
    grid=(_GRID,),
    in_specs=[
        pl.BlockSpec((1, _HIDDEN), lambda j: (0, 0)),
        pl.BlockSpec(memory_space=pltpu.SMEM),
        pl.BlockSpec((_HIDDEN, _BLK), lambda j: (0, j)),
    ],
    out_specs=pl.BlockSpec((_BLK,), lambda j: (j,)),
    out_shape=jax.ShapeDtypeStruct((_NITEMS,), jnp.float32),
)

_mesh = plsc.VectorSubcoreMesh(core_axis_name="c", subcore_axis_name="s")


@functools.partial(
    pl.kernel,
    mesh=_mesh,
    out_type=jax.ShapeDtypeStruct((_BATCH,), jnp.float32),
    scratch_types=[
        pltpu.VMEM((_NCHUNK, _CHUNK), jnp.int32),
        pltpu.VMEM((_BPW,), jnp.float32),
        pltpu.SemaphoreType.DMA,
    ],
)
def _sc_gather(ids_hbm, proj_hbm, out_hbm, idx_v, vals_v, sem):
    wid = lax.axis_index("s") * _NC + lax.axis_index("c")

    pltpu.sync_copy(ids_hbm.at[pl.ds(wid * _NCHUNK, _NCHUNK)], idx_v)

    copies = [
        pltpu.async_copy(proj_hbm.at[idx_v.at[j]],
                         vals_v.at[pl.ds(j * _CHUNK, _CHUNK)], sem)
        for j in range(_NCHUNK)
    ]
    for c in copies:
        c.wait()

    pltpu.sync_copy(vals_v, out_hbm.at[pl.ds(wid * _BPW, _BPW)])


def kernel(item_ids, emb_table, fc_w, fc_b):
    ids2d = item_ids.astype(jnp.int32).reshape(_NW * _NCHUNK, _CHUNK)
    tabT = emb_table.T
    proj = _tc_project(fc_w.astype(jnp.float32),
                       fc_b.astype(jnp.float32), tabT)
    out = _sc_gather(ids2d, proj)
    return out.reshape(_BATCH, 1)


# speedup vs baseline: 1.5997x; 1.0011x over previous
import functools

import jax
import jax.numpy as jnp
from jax import lax
from jax.experimental import pallas as pl
from jax.experimental.pallas import tpu as pltpu
from jax.experimental.pallas import tpu_sc as plsc

_HIDDEN = 64
_BATCH = 16384
_NITEMS = 1000000
_NC, _NS, _L = 2, 16, 16
_NW = _NC * _NS
_BPW = _BATCH // _NW
_NCHUNK = 4
_CHUNK = _BPW // _NCHUNK

_BLK = 40960
_GRID = (_NITEMS + _BLK - 1) // _BLK


def _project_body(w_ref, b_ref, tabT_ref, out_ref):
    out_ref[...] = jnp.dot(w_ref[...], tabT_ref[...],
                           preferred_element_type=jnp.float32)[0] + b_ref[0]


_tc_project = pl.pallas_call(
    _project_body,
    grid=(_GRID,),
    in_specs=[
        pl.BlockSpec((1, _HIDDEN), lambda j: (0, 0)),
        pl.BlockSpec(memory_space=pltpu.SMEM),
        pl.BlockSpec((_HIDDEN, _BLK), lambda j: (0, j)),
    ],
    out_specs=pl.BlockSpec((_BLK,), lambda j: (j,)),
    out_shape=jax.ShapeDtypeStruct((_NITEMS,), jnp.float32),
)

_mesh = plsc.VectorSubcoreMesh(core_axis_name="c", subcore_axis_name="s")


@functools.partial(
    pl.kernel,
    mesh=_mesh,
    out_type=jax.ShapeDtypeStruct((_BATCH,), jnp.float32),
    scratch_types=[
        pltpu.VMEM((_NCHUNK, _CHUNK), jnp.int32),
        pltpu.VMEM((_BPW,), jnp.float32),
        pltpu.SemaphoreType.DMA,
    ],
)
def _sc_gather(ids_hbm, proj_hbm, out_hbm, idx_v, vals_v, sem):
    wid = lax.axis_index("s") * _NC + lax.axis_index("c")

    pltpu.sync_copy(ids_hbm.at[pl.ds(wid * _NCHUNK, _NCHUNK)], idx_v)

    copies = [
        pltpu.async_copy(proj_hbm.at[idx_v.at[j]],
                         vals_v.at[pl.ds(j * _CHUNK, _CHUNK)], sem)
        for j in range(_NCHUNK)
    ]
    for c in copies:
        c.wait()

    pltpu.sync_copy(vals_v, out_hbm.at[pl.ds(wid * _BPW, _BPW)])


def kernel(item_ids, emb_table, fc_w, fc_b):
    ids2d = item_ids.astype(jnp.int32).reshape(_NW * _NCHUNK, _CHUNK)
    tabT = emb_table.T
    proj = _tc_project(fc_w.astype(jnp.float32),
                       fc_b.astype(jnp.float32), tabT)
    out = _sc_gather(ids2d, proj)
    return out.reshape(_BATCH, 1)


# speedup vs baseline: 1.6070x; 1.0046x over previous
import functools

import jax
import jax.numpy as jnp
from jax import lax
from jax.experimental import pallas as pl
from jax.experimental.pallas import tpu as pltpu
from jax.experimental.pallas import tpu_sc as plsc

_HIDDEN = 64
_BATCH = 16384
_NITEMS = 1000000
_NC, _NS, _L = 2, 16, 16
_NW = _NC * _NS
_BPW = _BATCH // _NW
_NCHUNK = 4
_CHUNK = _BPW // _NCHUNK

_BLK = 32768
_GRID = (_NITEMS + _BLK - 1) // _BLK


def _project_body(w_ref, b_ref, tabT_ref, out_ref):
    out_ref[...] = jnp.dot(w_ref[...], tabT_ref[...],
                           preferred_element_type=jnp.float32)[0] + b_ref[0]


_tc_project = pl.pallas_call(
    _project_body,
    grid=(_GRID,),
    in_specs=[
        pl.BlockSpec((1, _HIDDEN), lambda j: (0, 0)),
        pl.BlockSpec(memory_space=pltpu.SMEM),
        pl.BlockSpec((_HIDDEN, _BLK), lambda j: (0, j)),
    ],
    out_specs=pl.BlockSpec((_BLK,), lambda j: (j,)),
    out_shape=jax.ShapeDtypeStruct((_NITEMS,), jnp.float32),
)

_mesh = plsc.VectorSubcoreMesh(core_axis_name="c", subcore_axis_name="s")


@functools.partial(
    pl.kernel,
    mesh=_mesh,
    out_type=jax.ShapeDtypeStruct((_BATCH,), jnp.float32),
    scratch_types=[
        pltpu.VMEM((_NCHUNK, _CHUNK), jnp.int32),
        pltpu.VMEM((_BPW,), jnp.float32),
        pltpu.SemaphoreType.DMA,
    ],
)
def _sc_gather(ids_hbm, proj_hbm, out_hbm, idx_v, vals_v, sem):
    wid = lax.axis_index("s") * _NC + lax.axis_index("c")

    pltpu.sync_copy(ids_hbm.at[pl.ds(wid * _NCHUNK, _NCHUNK)], idx_v)

    copies = [
        pltpu.async_copy(proj_hbm.at[idx_v.at[j]],
                         vals_v.at[pl.ds(j * _CHUNK, _CHUNK)], sem)
        for j in range(_NCHUNK)
    ]
    for c in copies:
        c.wait()

    pltpu.sync_copy(vals_v, out_hbm.at[pl.ds(wid * _BPW, _BPW)])


def kernel(item_ids, emb_table, fc_w, fc_b):
    ids2d = item_ids.astype(jnp.int32).reshape(_NW * _NCHUNK, _CHUNK)
    tabT = emb_table.T
    proj = _tc_project(fc_w.astype(jnp.float32),
                       fc_b.astype(jnp.float32), tabT)
    out = _sc_gather(ids2d, proj)
    return out.reshape(_BATCH, 1)
